# Initial kernel scaffold; baseline (speedup 1.0000x reference)
import jax
import jax.numpy as jnp
from jax.experimental import pallas as pl


def kernel(x, prev_q, codes):
    raise NotImplementedError("write your pallas kernel here")


# speedup vs baseline: 2.7344x; 2.7344x over previous
import jax
import jax.numpy as jnp
from jax.experimental import pallas as pl

_ALPHA = 0.1
_BETA = 0.25
_EPS = 1e-06
_TB = 64


def _tc_body(x_ref, pq_ref, codes_ref, codesT_ref, q_ref, idx_ref, loss_ref):
    i = pl.program_id(0)
    nsteps = pl.num_programs(0)
    xb = x_ref[...]
    pq = pq_ref[...]
    codes = codes_ref[...]
    codesT = codesT_ref[...]
    D = xb.shape[1]
    K = codes.shape[0]
    rsqrt_d = 1.0 / (D ** 0.5)

    norm = jnp.sqrt(jnp.sum(pq * pq, axis=1, keepdims=True))
    u = pq / jnp.maximum(norm, 1e-6)
    p = u * rsqrt_d
    c = jnp.sum(p, axis=1, keepdims=True)
    t = (1.0 + c) + _EPS

    A = p[:, :, None] - p[:, None, :]
    Abf = A.astype(jnp.bfloat16)
    A2 = jax.lax.dot_general(
        Abf, Abf, (((2,), (1,)), ((0,), (0,))),
        preferred_element_type=jnp.float32)

    ii = jax.lax.broadcasted_iota(jnp.int32, (1, D, D), 1)
    jj = jax.lax.broadcasted_iota(jnp.int32, (1, D, D), 2)
    eye = (ii == jj).astype(jnp.float32)
    R = eye + A + A2 / t[:, :, None]
    Rbf = R.astype(jnp.bfloat16)
    xbf = xb.astype(jnp.bfloat16)

    x_c = jax.lax.dot_general(
        xbf, Rbf, (((1,), (1,)), ((0,), (0,))),
        preferred_element_type=jnp.float32)

    xc2 = jnp.sum(x_c * x_c, axis=1, keepdims=True)
    cn2 = jnp.sum(codesT * codesT, axis=0, keepdims=True)
    scores = (xc2 + cn2) - 2.0 * jnp.dot(
        x_c, codesT, precision=jax.lax.Precision.HIGHEST)
    m = jnp.min(scores, axis=1, keepdims=True)
    kiota = jax.lax.broadcasted_iota(jnp.int32, scores.shape, 1)
    idx = jnp.min(jnp.where(scores == m, kiota, K),
                  axis=1, keepdims=True)
    idx_ref[...] = idx.astype(jnp.int32)

    onehot = (kiota == idx).astype(jnp.float32)
    q_c = jnp.dot(onehot, codes, precision=jax.lax.Precision.HIGHEST)

    qbf = q_c.astype(jnp.bfloat16)
    quantized = jax.lax.dot_general(
        qbf, Rbf, (((1,), (2,)), ((0,), (0,))),
        preferred_element_type=jnp.float32)
    q_ref[...] = quantized

    diff = xb - quantized
    partial = jnp.sum(jnp.sum(diff * diff, axis=1, keepdims=True),
                      axis=0, keepdims=True)

    @pl.when(i == 0)
    def _():
        loss_ref[...] = jnp.zeros_like(loss_ref)

    loss_ref[...] += partial

    @pl.when(i == nsteps - 1)
    def _():
        B_total = nsteps * xb.shape[0]
        loss_ref[...] = loss_ref[...] * ((1.0 + _BETA) / B_total)


@jax.jit
def kernel(x, prev_q, codes):
    B, D = x.shape
    codes2 = codes[0]
    K = codes2.shape[0]
    codesT = codes2.T
    grid = (B // _TB,)
    q, idx, loss = pl.pallas_call(
        _tc_body,
        grid=grid,
        in_specs=[
            pl.BlockSpec((_TB, D), lambda i: (i, 0)),
            pl.BlockSpec((_TB, D), lambda i: (i, 0)),
            pl.BlockSpec((K, D), lambda i: (0, 0)),
            pl.BlockSpec((D, K), lambda i: (0, 0)),
        ],
        out_specs=[
            pl.BlockSpec((_TB, D), lambda i: (i, 0)),
            pl.BlockSpec((_TB, 1), lambda i: (i, 0)),
            pl.BlockSpec((1, 1), lambda i: (0, 0)),
        ],
        out_shape=[
            jax.ShapeDtypeStruct((B, D), jnp.float32),
            jax.ShapeDtypeStruct((B, 1), jnp.int32),
            jax.ShapeDtypeStruct((1, 1), jnp.float32),
        ],
    )(x, prev_q, codes2, codesT)
    return q, jnp.reshape(idx, (B,)), jnp.reshape(loss, ())


# speedup vs baseline: 3.5166x; 1.2860x over previous
import jax
import jax.numpy as jnp
from jax.experimental import pallas as pl

_ALPHA = 0.1
_BETA = 0.25
_EPS = 1e-06
_TB = 128


def _tc_body(x_ref, pq_ref, codes_ref, codesT_ref, q_ref, idx_ref, loss_ref):
    i = pl.program_id(0)
    nsteps = pl.num_programs(0)
    xb = x_ref[...]
    pq = pq_ref[...]
    codes = codes_ref[...]
    codesT = codesT_ref[...]
    D = xb.shape[1]
    K = codes.shape[0]
    rsqrt_d = 1.0 / (D ** 0.5)

    norm = jnp.sqrt(jnp.sum(pq * pq, axis=1, keepdims=True))
    u = pq / jnp.maximum(norm, 1e-6)
    p = u * rsqrt_d
    c = jnp.sum(p, axis=1, keepdims=True)
    t = (1.0 + c) + _EPS

    A = p[:, :, None] - p[:, None, :]
    Abf = A.astype(jnp.bfloat16)
    A2 = jax.lax.dot_general(
        Abf, Abf, (((2,), (1,)), ((0,), (0,))),
        preferred_element_type=jnp.float32)

    ii = jax.lax.broadcasted_iota(jnp.int32, (1, D, D), 1)
    jj = jax.lax.broadcasted_iota(jnp.int32, (1, D, D), 2)
    eye = (ii == jj).astype(jnp.float32)
    R = eye + A + A2 / t[:, :, None]
    Rbf = R.astype(jnp.bfloat16)
    xbf = xb.astype(jnp.bfloat16)

    x_c = jax.lax.dot_general(
        xbf, Rbf, (((1,), (1,)), ((0,), (0,))),
        preferred_element_type=jnp.float32)

    xc2 = jnp.sum(x_c * x_c, axis=1, keepdims=True)
    cn2 = jnp.sum(codesT * codesT, axis=0, keepdims=True)
    scores = (xc2 + cn2) - 2.0 * jnp.dot(
        x_c, codesT, precision=jax.lax.Precision.HIGHEST)
    m = jnp.min(scores, axis=1, keepdims=True)
    kiota = jax.lax.broadcasted_iota(jnp.int32, scores.shape, 1)
    idx = jnp.min(jnp.where(scores == m, kiota, K),
                  axis=1, keepdims=True)
    idx_ref[...] = idx.astype(jnp.int32)

    onehot = (kiota == idx).astype(jnp.bfloat16)
    qbf = jax.lax.dot_general(
        onehot, codes, (((1,), (0,)), ((), ())),
        preferred_element_type=jnp.float32).astype(jnp.bfloat16)

    quantized = jax.lax.dot_general(
        qbf, Rbf, (((1,), (2,)), ((0,), (0,))),
        preferred_element_type=jnp.float32)
    q_ref[...] = quantized

    diff = xb - quantized
    partial = jnp.sum(jnp.sum(diff * diff, axis=1, keepdims=True),
                      axis=0, keepdims=True)

    @pl.when(i == 0)
    def _():
        loss_ref[...] = jnp.zeros_like(loss_ref)

    loss_ref[...] += partial

    @pl.when(i == nsteps - 1)
    def _():
        B_total = nsteps * xb.shape[0]
        loss_ref[...] = loss_ref[...] * ((1.0 + _BETA) / B_total)


@jax.jit
def kernel(x, prev_q, codes):
    B, D = x.shape
    codes2 = codes[0]
    K = codes2.shape[0]
    codesT = codes2.T
    codes_bf = codes2.astype(jnp.bfloat16)
    grid = (B // _TB,)
    q, idx, loss = pl.pallas_call(
        _tc_body,
        grid=grid,
        in_specs=[
            pl.BlockSpec((_TB, D), lambda i: (i, 0)),
            pl.BlockSpec((_TB, D), lambda i: (i, 0)),
            pl.BlockSpec((K, D), lambda i: (0, 0)),
            pl.BlockSpec((D, K), lambda i: (0, 0)),
        ],
        out_specs=[
            pl.BlockSpec((_TB, D), lambda i: (i, 0)),
            pl.BlockSpec((_TB, 1), lambda i: (i, 0)),
            pl.BlockSpec((1, 1), lambda i: (0, 0)),
        ],
        out_shape=[
            jax.ShapeDtypeStruct((B, D), jnp.float32),
            jax.ShapeDtypeStruct((B, 1), jnp.int32),
            jax.ShapeDtypeStruct((1, 1), jnp.float32),
        ],
    )(x, prev_q, codes_bf, codesT)
    return q, jnp.reshape(idx, (B,)), jnp.reshape(loss, ())
